# Initial kernel scaffold; baseline (speedup 1.0000x reference)
import jax
import jax.numpy as jnp
from jax.experimental import pallas as pl


def kernel(position_ids, table):
    raise NotImplementedError("write your pallas kernel here")


# speedup vs baseline: 1.9481x; 1.9481x over previous
import functools

import jax
import jax.numpy as jnp
from jax import lax
from jax.experimental import pallas as pl
from jax.experimental.pallas import tpu as pltpu
from jax.experimental.pallas import tpu_sc as plsc

_INFO = plsc.get_sparse_core_info()
_NC = _INFO.num_cores
_NS = _INFO.num_subcores
_NW = _NC * _NS


@functools.lru_cache(maxsize=None)
def _make_gather(B, D, C):
  assert B % (_NW * C) == 0 and C % 8 == 0 and C <= 128
  per_worker = B // _NW
  n_chunks = per_worker // C
  mesh = plsc.VectorSubcoreMesh(core_axis_name="c", subcore_axis_name="s")

  @functools.partial(
      pl.kernel,
      out_type=jax.ShapeDtypeStruct((B, D), jnp.float32),
      mesh=mesh,
      scratch_types=[
          pltpu.VMEM((C,), jnp.int32),
          pltpu.VMEM((C, D), jnp.float32),
          pltpu.SemaphoreType.DMA,
      ],
  )
  def gather(ids_hbm, table_hbm, out_hbm, idx_v, rows_v, sem):
    wid = lax.axis_index("s") * _NC + lax.axis_index("c")
    base = wid * per_worker

    def body(j, carry):
      off = base + j * C
      pltpu.sync_copy(ids_hbm.at[pl.ds(off, C)], idx_v)
      pltpu.async_copy(table_hbm.at[idx_v], rows_v, sem).wait()
      pltpu.sync_copy(rows_v, out_hbm.at[pl.ds(off, C)])
      return carry

    lax.fori_loop(0, n_chunks, body, 0)

  return gather


def kernel(position_ids, table):
  batch, seq = position_ids.shape
  d = table.shape[1]
  ids = position_ids.reshape(-1).astype(jnp.int32)
  out = _make_gather(ids.shape[0], d, 64)(ids, table)
  return out.reshape(batch, seq, d)


# speedup vs baseline: 2.0694x; 1.0622x over previous
import functools

import jax
import jax.numpy as jnp
from jax import lax
from jax.experimental import pallas as pl
from jax.experimental.pallas import tpu as pltpu
from jax.experimental.pallas import tpu_sc as plsc

_INFO = plsc.get_sparse_core_info()
_NC = _INFO.num_cores
_NS = _INFO.num_subcores
_NW = _NC * _NS


@functools.lru_cache(maxsize=None)
def _make_gather(B, D, C):
  assert B % (_NW * C) == 0 and C % 8 == 0 and C <= 128
  per_worker = B // _NW
  n_chunks = per_worker // C
  mesh = plsc.VectorSubcoreMesh(core_axis_name="c", subcore_axis_name="s")

  @functools.partial(
      pl.kernel,
      out_type=jax.ShapeDtypeStruct((B, D), jnp.float32),
      mesh=mesh,
      scratch_types=[
          pltpu.VMEM((n_chunks, C), jnp.int32),
          pltpu.VMEM((C, D), jnp.float32),
          pltpu.VMEM((C, D), jnp.float32),
          pltpu.SemaphoreType.DMA,
          pltpu.SemaphoreType.DMA,
          pltpu.SemaphoreType.DMA,
          pltpu.SemaphoreType.DMA,
      ],
  )
  def gather(ids_hbm, table_hbm, out_hbm, idx_v, rows0, rows1, g0, g1, s0,
             s1):
    wid = lax.axis_index("s") * _NC + lax.axis_index("c")
    base = wid * per_worker
    rows = (rows0, rows1)
    gsem = (g0, g1)
    ssem = (s0, s1)

    pltpu.sync_copy(ids_hbm.at[pl.ds(wid * n_chunks, n_chunks)], idx_v)

    def fire_gather(j):
      b = j % 2
      return pltpu.async_copy(table_hbm.at[idx_v.at[j]], rows[b], gsem[b])

    def fire_store(j):
      b = j % 2
      return pltpu.async_copy(rows[b], out_hbm.at[pl.ds(base + j * C, C)],
                              ssem[b])

    gd = [None, None]
    sd = [None, None]
    gd[0] = fire_gather(0)
    gd[1] = fire_gather(1)
    for j in range(n_chunks):
      b = j % 2
      gd[b].wait()
      sd[b] = fire_store(j)
      if j + 2 < n_chunks:
        sd[b].wait()
        gd[b] = fire_gather(j + 2)
    sd[(n_chunks - 2) % 2].wait()
    sd[(n_chunks - 1) % 2].wait()

  return gather


def kernel(position_ids, table):
  batch, seq = position_ids.shape
  d = table.shape[1]
  C = 32
  ids = position_ids.reshape(-1, C).astype(jnp.int32)
  out = _make_gather(batch * seq, d, C)(ids, table)
  return out.reshape(batch, seq, d)
